# Initial kernel scaffold; baseline (speedup 1.0000x reference)
import jax
import jax.numpy as jnp
from jax.experimental import pallas as pl


def kernel(x, edge_index, edge_attr, batch, W_ne, b_ne, ln_g, ln_b, W_ee, b_ee, W1, We1, as1, ad1, ae1, b1, W2, We2, as2, ad2, ae2, b2, Wg, bg, Wq1, bq1, Wq2, bq2):
    raise NotImplementedError("write your pallas kernel here")


# speedup vs baseline: 24.5367x; 24.5367x over previous
import functools

import jax
import jax.numpy as jnp
from jax import lax
from jax.experimental import pallas as pl
from jax.experimental.pallas import tpu as pltpu
from jax.experimental.pallas import tpu_sc as plsc

N = 10000
E = 320000
DF = 128
DE = 16
H = 128

NP = 10240
NC, NS = 2, 16
NW = NC * NS
RW = 80
NR = E // RW
SROWS = NR // NS
SCH = 25
WROWS = NR // NW
NBUF = 3
NZT = NP // NS
NPR = NP // 16
DDZ = NPR // NS


def _prep_body(x_ref, wne_ref, bne_ref, g_ref, b_ref, w1_ref, as_ref, ad_ref,
               hw_ref, s_ref, d_ref):
    h = jnp.maximum(
        jnp.dot(x_ref[...], wne_ref[...], preferred_element_type=jnp.float32)
        + bne_ref[...][None, :], 0.0)
    mu = jnp.mean(h, axis=1, keepdims=True)
    var = jnp.mean((h - mu) ** 2, axis=1, keepdims=True)
    h = (h - mu) / jnp.sqrt(var + 1e-5) * g_ref[...][None, :] + b_ref[...][None, :]
    hw = jnp.dot(h, w1_ref[...], preferred_element_type=jnp.float32)
    hw_ref[...] = hw
    s_ref[...] = jnp.dot(hw, as_ref[...][:, None], preferred_element_type=jnp.float32)
    d_ref[...] = jnp.dot(hw, ad_ref[...][:, None], preferred_element_type=jnp.float32)


def _tc_prep(x, wne, bne, g, b, w1, a_s, a_d):
    return pl.pallas_call(
        _prep_body,
        out_shape=[jax.ShapeDtypeStruct((N, H), jnp.float32),
                   jax.ShapeDtypeStruct((N, 1), jnp.float32),
                   jax.ShapeDtypeStruct((N, 1), jnp.float32)],
    )(x, wne, bne, g, b, w1, a_s, a_d)


def _mid_body(p_ref, b1_ref, w2_ref, as_ref, ad_ref, hw_ref, s_ref, d_ref):
    h = jnp.maximum(p_ref[0] + p_ref[1] + b1_ref[...][None, :], 0.0)
    hw = jnp.dot(h, w2_ref[...], preferred_element_type=jnp.float32)
    hw_ref[...] = hw
    s_ref[...] = jnp.dot(hw, as_ref[...][:, None], preferred_element_type=jnp.float32)
    d_ref[...] = jnp.dot(hw, ad_ref[...][:, None], preferred_element_type=jnp.float32)


def _tc_mid(partials, b1, w2, a_s, a_d):
    return pl.pallas_call(
        _mid_body,
        out_shape=[jax.ShapeDtypeStruct((N, H), jnp.float32),
                   jax.ShapeDtypeStruct((N, 1), jnp.float32),
                   jax.ShapeDtypeStruct((N, 1), jnp.float32)],
    )(partials, b1, w2, a_s, a_d)


EB = 16000


def _edge_body(ea_ref, wee_ref, bee_ref, we1_ref, ae1_ref, we2_ref, ae2_ref,
               o1_ref, o2_ref):
    v1 = jnp.dot(we1_ref[...], ae1_ref[...][:, None],
                 preferred_element_type=jnp.float32)
    v2 = jnp.dot(we2_ref[...], ae2_ref[...][:, None],
                 preferred_element_type=jnp.float32)
    e = jnp.maximum(
        jnp.dot(ea_ref[...], wee_ref[...], preferred_element_type=jnp.float32)
        + bee_ref[...][None, :], 0.0)
    o1_ref[...] = jnp.dot(e, v1, preferred_element_type=jnp.float32)
    o2_ref[...] = jnp.dot(e, v2, preferred_element_type=jnp.float32)


def _tc_edge(edge_attr, wee, bee, we1, ae1, we2, ae2):
    nb = E // EB
    o1, o2 = pl.pallas_call(
        _edge_body,
        grid=(nb,),
        in_specs=[
            pl.BlockSpec((EB, DE), lambda i: (i, 0)),
            pl.BlockSpec((DE, H), lambda i: (0, 0)),
            pl.BlockSpec((H,), lambda i: (0,)),
            pl.BlockSpec((H, H), lambda i: (0, 0)),
            pl.BlockSpec((H,), lambda i: (0,)),
            pl.BlockSpec((H, H), lambda i: (0, 0)),
            pl.BlockSpec((H,), lambda i: (0,)),
        ],
        out_specs=[pl.BlockSpec((EB, 1), lambda i: (i, 0)),
                   pl.BlockSpec((EB, 1), lambda i: (i, 0))],
        out_shape=[jax.ShapeDtypeStruct((E, 1), jnp.float32),
                   jax.ShapeDtypeStruct((E, 1), jnp.float32)],
    )(edge_attr, wee, bee, we1, ae1, we2, ae2)
    return o1.reshape(E), o2.reshape(E)


def _final_body(p_ref, b2_ref, wg_ref, bg_ref, wq1_ref, bq1_ref, wq2_ref,
                bq2_ref, q_ref):
    h = jnp.maximum(p_ref[0] + p_ref[1] + b2_ref[...][None, :], 0.0)
    gate = (jnp.dot(h, wg_ref[...], preferred_element_type=jnp.float32)
            + bg_ref[...][None, :])[:, 0]
    gm = jnp.max(gate)
    gex = jnp.exp(gate - gm)
    gden = jnp.sum(gex)
    coef = gex / (gden + 1e-16)
    pooled = jnp.dot(coef[None, :], h, preferred_element_type=jnp.float32)
    z = (jnp.dot(h, wq1_ref[0:H, :], preferred_element_type=jnp.float32)
         + jnp.dot(pooled, wq1_ref[H:2 * H, :], preferred_element_type=jnp.float32)
         + bq1_ref[...][None, :])
    z = jnp.maximum(z, 0.0)
    q_ref[...] = (jnp.dot(z, wq2_ref[...], preferred_element_type=jnp.float32)
                  + bq2_ref[...][None, :])


def _tc_final(partials, b2, wg, bg, wq1, bq1, wq2, bq2):
    return pl.pallas_call(
        _final_body,
        out_shape=jax.ShapeDtypeStruct((N, 1), jnp.float32),
    )(partials, b2, wg, bg, wq1, bq1, wq2, bq2)


def _coef_sc_body(src_hbm, dst3_hbm, ea_hbm, s_hbm, d_hbm, coef_hbm,
                  s_v, d_v, den_v, esrc_v, edst_v, eea_v, ccoef_v,
                  ztmp_v, idxb_v, dden_shr):
    cid = lax.axis_index("c")
    sid = lax.axis_index("s")
    wid = sid * NC + cid
    zero16 = jnp.zeros((16,), jnp.float32)

    pltpu.sync_copy(s_hbm, s_v)
    pltpu.sync_copy(d_hbm, d_v)

    def zden(i, _):
        den_v[i, pl.ds(0, 16)] = zero16
        return 0
    lax.fori_loop(0, NPR, zden, 0, unroll=8)

    def zt(i, _):
        ztmp_v[i, pl.ds(0, 16)] = zero16
        return 0
    lax.fori_loop(0, DDZ, zt, 0, unroll=8)
    pltpu.sync_copy(ztmp_v, dden_shr.at[pl.ds(sid * DDZ, DDZ)])
    plsc.subcore_barrier()

    def sc_major(mj, _):
        m = NC * sid + mj

        def sc_chunk(ci, _):
            e0 = (m * WROWS + ci * SCH) * RW
            pltpu.sync_copy(src_hbm.at[pl.ds(e0, SCH * RW)], esrc_v)
            pltpu.sync_copy(dst3_hbm.at[m, pl.ds(ci * SCH, SCH)], edst_v)
            pltpu.sync_copy(ea_hbm.at[pl.ds(e0, SCH * RW)], eea_v)

            def rowloop(r, _):
                for k in range(RW // 16):
                    o = (r * (RW // 16) + k) * 16
                    src16 = esrc_v[pl.ds(o, 16)]
                    dst16 = edst_v.at[r][pl.ds(k * 16, 16)]
                    a = (plsc.load_gather(s_v, [src16])
                         + plsc.load_gather(d_v, [dst16])
                         + eea_v[pl.ds(o, 16)])
                    a = jnp.maximum(a, 0.2 * a)
                    plsc.addupdate_scatter(
                        den_v,
                        [jnp.right_shift(dst16, 4), jnp.bitwise_and(dst16, 15)],
                        jnp.exp(a))
                return 0
            lax.fori_loop(0, SCH, rowloop, 0)
            return 0
        lax.fori_loop(0, WROWS // SCH, sc_chunk, 0)
        return 0
    lax.fori_loop(0, NR // (NS * WROWS), sc_major, 0)

    iota16 = lax.iota(jnp.int32, 16)
    for c in range(NPR // 128):
        for g in range(8):
            idxb_v[pl.ds(g * 16, 16)] = iota16 + (c * 128 + g * 16)
        pltpu.sync_copy(den_v.at[pl.ds(c * 128, 128)],
                        dden_shr.at[idxb_v], add=True)
    plsc.subcore_barrier()
    pltpu.sync_copy(dden_shr, den_v)

    def cf_chunk(ci, _):
        e0 = (wid * WROWS + ci * SCH) * RW
        pltpu.sync_copy(src_hbm.at[pl.ds(e0, SCH * RW)], esrc_v)
        pltpu.sync_copy(dst3_hbm.at[wid, pl.ds(ci * SCH, SCH)], edst_v)
        pltpu.sync_copy(ea_hbm.at[pl.ds(e0, SCH * RW)], eea_v)

        def rowloop(r, _):
            for k in range(RW // 16):
                o = (r * (RW // 16) + k) * 16
                src16 = esrc_v[pl.ds(o, 16)]
                dst16 = edst_v.at[r][pl.ds(k * 16, 16)]
                a = (plsc.load_gather(s_v, [src16])
                     + plsc.load_gather(d_v, [dst16])
                     + eea_v[pl.ds(o, 16)])
                a = jnp.maximum(a, 0.2 * a)
                den16 = plsc.load_gather(
                    den_v,
                    [jnp.right_shift(dst16, 4), jnp.bitwise_and(dst16, 15)])
                ccoef_v[pl.ds(o, 16)] = jnp.exp(a) / (den16 + 1e-16)
            return 0
        lax.fori_loop(0, SCH, rowloop, 0)
        pltpu.sync_copy(ccoef_v, coef_hbm.at[pl.ds(e0, SCH * RW)])
        return 0
    lax.fori_loop(0, WROWS // SCH, cf_chunk, 0)


def _coef_sc(src, dst3, ea, s, d):
    mesh = plsc.VectorSubcoreMesh(core_axis_name="c", subcore_axis_name="s",
                                  num_cores=NC, num_subcores=NS)
    call = pl.kernel(
        _coef_sc_body,
        out_type=jax.ShapeDtypeStruct((E,), jnp.float32),
        mesh=mesh,
        compiler_params=pltpu.CompilerParams(needs_layout_passes=False,
                                             use_tc_tiling_on_sc=False),
        scratch_types=[
            pltpu.VMEM((N,), jnp.float32),
            pltpu.VMEM((N,), jnp.float32),
            pltpu.VMEM((NPR, 16), jnp.float32),
            pltpu.VMEM((SCH * RW,), jnp.int32),
            pltpu.VMEM((SCH, RW), jnp.int32),
            pltpu.VMEM((SCH * RW,), jnp.float32),
            pltpu.VMEM((SCH * RW,), jnp.float32),
            pltpu.VMEM((DDZ, 16), jnp.float32),
            pltpu.VMEM((128,), jnp.int32),
            pltpu.VMEM_SHARED((NPR, 16), jnp.float32),
        ],
    )
    return call(src, dst3, ea, s, d)


def _scat_sc_body(src_hbm, dst3_hbm, coef_hbm, hw_hbm, out_hbm,
                  wdst2_v, srcr_v, coefr_v, rows_v,
                  acc_shr, semg, sems, semi):
    cid = lax.axis_index("c")
    sid = lax.axis_index("s")
    wid = sid * NC + cid
    zero16 = jnp.zeros((16,), jnp.float32)
    nbase = sid * NZT

    pltpu.sync_copy(dst3_hbm.at[wid], wdst2_v)

    def zrow(i, _):
        row = rows_v.at[0].at[i]
        for j in range(H // 16):
            row[pl.ds(j * 16, 16)] = zero16
        return 0
    lax.fori_loop(0, RW, zrow, 0)
    for p in range(NZT // RW):
        pltpu.sync_copy(rows_v.at[0], acc_shr.at[pl.ds(nbase + p * RW, RW)])
    plsc.subcore_barrier()

    ebase = wid * WROWS * RW

    def fire_smalls(r, b):
        pltpu.async_copy(src_hbm.at[pl.ds(ebase + r * RW, RW)],
                         srcr_v.at[b], semi.at[b])
        pltpu.async_copy(coef_hbm.at[pl.ds(ebase + r * RW, RW)],
                         coefr_v.at[b], semi.at[b])

    def wait_smalls(r, b):
        pltpu.make_async_copy(src_hbm.at[pl.ds(ebase + r * RW, RW)],
                              srcr_v.at[b], semi.at[b]).wait()
        pltpu.make_async_copy(coef_hbm.at[pl.ds(ebase + r * RW, RW)],
                              coefr_v.at[b], semi.at[b]).wait()

    def fire_gather(r, b):
        pltpu.async_copy(hw_hbm.at[srcr_v.at[b]], rows_v.at[b], semg.at[b])

    def wait_gather(r, b):
        pltpu.make_async_copy(hw_hbm.at[srcr_v.at[b]], rows_v.at[b],
                              semg.at[b]).wait()

    def fire_scatter(r, b):
        pltpu.async_copy(rows_v.at[b], acc_shr.at[wdst2_v.at[r]],
                         sems.at[b], add=True)

    def wait_scatter(r, b):
        pltpu.make_async_copy(rows_v.at[b], acc_shr.at[wdst2_v.at[r]],
                              sems.at[b]).wait()

    def step(r):
        b = r % NBUF

        @pl.when(r >= 2)
        def _():
            wait_scatter(r - 2, (r + 1) % NBUF)

        @pl.when(r + 2 < WROWS)
        def _():
            fire_smalls(r + 2, (r + 2) % NBUF)

        @pl.when(r + 1 < WROWS)
        def _():
            wait_smalls(r + 1, (r + 1) % NBUF)
            fire_gather(r + 1, (r + 1) % NBUF)

        wait_gather(r, b)

        def scale(e, _):
            c16 = plsc.load_gather(coefr_v.at[b],
                                   [jnp.full((16,), e, jnp.int32)])
            row = rows_v.at[b].at[e]
            for j in range(H // 16):
                row[pl.ds(j * 16, 16)] = row[pl.ds(j * 16, 16)] * c16
            return 0
        lax.fori_loop(0, RW, scale, 0)
        fire_scatter(r, b)

    fire_smalls(0, 0)
    fire_smalls(1, 1)
    wait_smalls(0, 0)
    fire_gather(0, 0)

    def steps3(i, _):
        for bb in range(NBUF):
            step(i * NBUF + bb)
        return 0
    nfull = (WROWS // NBUF) * NBUF
    lax.fori_loop(0, WROWS // NBUF, steps3, 0)
    for r in range(nfull, WROWS):
        step(r)
    for r in range(WROWS - 2, WROWS):
        wait_scatter(r, r % NBUF)
    plsc.subcore_barrier()

    for p in range(NZT // RW):
        pltpu.sync_copy(acc_shr.at[pl.ds(nbase + p * RW, RW)], rows_v.at[0])
        pltpu.sync_copy(rows_v.at[0], out_hbm.at[cid, pl.ds(nbase + p * RW, RW)])


def _scat_sc(src, dst3, coef, hw):
    mesh = plsc.VectorSubcoreMesh(core_axis_name="c", subcore_axis_name="s",
                                  num_cores=NC, num_subcores=NS)
    call = pl.kernel(
        _scat_sc_body,
        out_type=jax.ShapeDtypeStruct((NC, NP, H), jnp.float32),
        mesh=mesh,
        compiler_params=pltpu.CompilerParams(needs_layout_passes=False,
                                             use_tc_tiling_on_sc=False),
        scratch_types=[
            pltpu.VMEM((WROWS, RW), jnp.int32),
            pltpu.VMEM((NBUF, RW), jnp.int32),
            pltpu.VMEM((NBUF, RW), jnp.float32),
            pltpu.VMEM((NBUF, RW, H), jnp.float32),
            pltpu.VMEM_SHARED((NP, H), jnp.float32),
            pltpu.SemaphoreType.DMA((NBUF,)),
            pltpu.SemaphoreType.DMA((NBUF,)),
            pltpu.SemaphoreType.DMA((NBUF,)),
        ],
    )
    return call(src, dst3, coef, hw)


def _gat_sc(src, dst3, ea, s, d, hw):
    coef = _coef_sc(src, dst3, ea, s, d)
    return _scat_sc(src, dst3, coef, hw)


def kernel(x, edge_index, edge_attr, batch, W_ne, b_ne, ln_g, ln_b, W_ee, b_ee,
           W1, We1, as1, ad1, ae1, b1, W2, We2, as2, ad2, ae2, b2, Wg, bg,
           Wq1, bq1, Wq2, bq2):
    src = edge_index[0]
    dst = edge_index[1]
    dst2 = dst.reshape(NW, WROWS, RW)

    hw1, s1, d1 = _tc_prep(x, W_ne, b_ne, ln_g, ln_b, W1, as1, ad1)
    ea1, ea2 = _tc_edge(edge_attr, W_ee, b_ee, We1, ae1, We2, ae2)

    p1 = _gat_sc(src, dst2, ea1, s1.reshape(N), d1.reshape(N), hw1)
    hw2, s2, d2 = _tc_mid(p1[:, :N, :], b1, W2, as2, ad2)

    p2 = _gat_sc(src, dst2, ea2, s2.reshape(N), d2.reshape(N), hw2)
    q = _tc_final(p2[:, :N, :], b2, Wg, bg, Wq1, bq1, Wq2, bq2)
    return q.reshape(N)


# speedup vs baseline: 27.4742x; 1.1197x over previous
import functools

import jax
import jax.numpy as jnp
from jax import lax
from jax.experimental import pallas as pl
from jax.experimental.pallas import tpu as pltpu
from jax.experimental.pallas import tpu_sc as plsc

N = 10000
E = 320000
DF = 128
DE = 16
H = 128

NP = 10240
NC, NS = 2, 16
NW = NC * NS
RW = 80
NR = E // RW
SROWS = NR // NS
SCH = 125
WROWS = NR // NW
NBUF = 3
NZT = NP // NS
NPR = NP // 16
DDZ = NPR // NS


def _prep_body(x_ref, wne_ref, bne_ref, g_ref, b_ref, w1_ref, as_ref, ad_ref,
               hw_ref, s_ref, d_ref):
    h = jnp.maximum(
        jnp.dot(x_ref[...], wne_ref[...], preferred_element_type=jnp.float32)
        + bne_ref[...][None, :], 0.0)
    mu = jnp.mean(h, axis=1, keepdims=True)
    var = jnp.mean((h - mu) ** 2, axis=1, keepdims=True)
    h = (h - mu) / jnp.sqrt(var + 1e-5) * g_ref[...][None, :] + b_ref[...][None, :]
    hw = jnp.dot(h, w1_ref[...], preferred_element_type=jnp.float32)
    hw_ref[...] = hw
    s_ref[...] = jnp.dot(hw, as_ref[...][:, None], preferred_element_type=jnp.float32)
    d_ref[...] = jnp.dot(hw, ad_ref[...][:, None], preferred_element_type=jnp.float32)


def _tc_prep(x, wne, bne, g, b, w1, a_s, a_d):
    return pl.pallas_call(
        _prep_body,
        out_shape=[jax.ShapeDtypeStruct((N, H), jnp.float32),
                   jax.ShapeDtypeStruct((N, 1), jnp.float32),
                   jax.ShapeDtypeStruct((N, 1), jnp.float32)],
    )(x, wne, bne, g, b, w1, a_s, a_d)


def _mid_body(p_ref, b1_ref, w2_ref, as_ref, ad_ref, hw_ref, s_ref, d_ref):
    h = jnp.maximum(p_ref[0] + p_ref[1] + b1_ref[...][None, :], 0.0)
    hw = jnp.dot(h, w2_ref[...], preferred_element_type=jnp.float32)
    hw_ref[...] = hw
    s_ref[...] = jnp.dot(hw, as_ref[...][:, None], preferred_element_type=jnp.float32)
    d_ref[...] = jnp.dot(hw, ad_ref[...][:, None], preferred_element_type=jnp.float32)


def _tc_mid(partials, b1, w2, a_s, a_d):
    return pl.pallas_call(
        _mid_body,
        out_shape=[jax.ShapeDtypeStruct((N, H), jnp.float32),
                   jax.ShapeDtypeStruct((N, 1), jnp.float32),
                   jax.ShapeDtypeStruct((N, 1), jnp.float32)],
    )(partials, b1, w2, a_s, a_d)


EB = 16000


def _edge_body(ea_ref, wee_ref, bee_ref, we1_ref, ae1_ref, we2_ref, ae2_ref,
               o1_ref, o2_ref):
    v1 = jnp.dot(we1_ref[...], ae1_ref[...][:, None],
                 preferred_element_type=jnp.float32)
    v2 = jnp.dot(we2_ref[...], ae2_ref[...][:, None],
                 preferred_element_type=jnp.float32)
    e = jnp.maximum(
        jnp.dot(ea_ref[...], wee_ref[...], preferred_element_type=jnp.float32)
        + bee_ref[...][None, :], 0.0)
    o1_ref[...] = jnp.dot(e, v1, preferred_element_type=jnp.float32)
    o2_ref[...] = jnp.dot(e, v2, preferred_element_type=jnp.float32)


def _tc_edge(edge_attr, wee, bee, we1, ae1, we2, ae2):
    nb = E // EB
    o1, o2 = pl.pallas_call(
        _edge_body,
        grid=(nb,),
        in_specs=[
            pl.BlockSpec((EB, DE), lambda i: (i, 0)),
            pl.BlockSpec((DE, H), lambda i: (0, 0)),
            pl.BlockSpec((H,), lambda i: (0,)),
            pl.BlockSpec((H, H), lambda i: (0, 0)),
            pl.BlockSpec((H,), lambda i: (0,)),
            pl.BlockSpec((H, H), lambda i: (0, 0)),
            pl.BlockSpec((H,), lambda i: (0,)),
        ],
        out_specs=[pl.BlockSpec((EB, 1), lambda i: (i, 0)),
                   pl.BlockSpec((EB, 1), lambda i: (i, 0))],
        out_shape=[jax.ShapeDtypeStruct((E, 1), jnp.float32),
                   jax.ShapeDtypeStruct((E, 1), jnp.float32)],
    )(edge_attr, wee, bee, we1, ae1, we2, ae2)
    return o1.reshape(E), o2.reshape(E)


def _final_body(p_ref, b2_ref, wg_ref, bg_ref, wq1_ref, bq1_ref, wq2_ref,
                bq2_ref, q_ref):
    h = jnp.maximum(p_ref[0] + p_ref[1] + b2_ref[...][None, :], 0.0)
    gate = (jnp.dot(h, wg_ref[...], preferred_element_type=jnp.float32)
            + bg_ref[...][None, :])[:, 0]
    gm = jnp.max(gate)
    gex = jnp.exp(gate - gm)
    gden = jnp.sum(gex)
    coef = gex / (gden + 1e-16)
    pooled = jnp.dot(coef[None, :], h, preferred_element_type=jnp.float32)
    z = (jnp.dot(h, wq1_ref[0:H, :], preferred_element_type=jnp.float32)
         + jnp.dot(pooled, wq1_ref[H:2 * H, :], preferred_element_type=jnp.float32)
         + bq1_ref[...][None, :])
    z = jnp.maximum(z, 0.0)
    q_ref[...] = (jnp.dot(z, wq2_ref[...], preferred_element_type=jnp.float32)
                  + bq2_ref[...][None, :])


def _tc_final(partials, b2, wg, bg, wq1, bq1, wq2, bq2):
    return pl.pallas_call(
        _final_body,
        out_shape=jax.ShapeDtypeStruct((N, 1), jnp.float32),
    )(partials, b2, wg, bg, wq1, bq1, wq2, bq2)


def _coef_sc_body(src_hbm, dst3_hbm, ea_hbm, s_hbm, d_hbm, coef_hbm,
                  s_v, d_v, den_v, esrc_v, edst_v, eea_v, ccoef_v,
                  ztmp_v, idxb_v, dden_shr):
    cid = lax.axis_index("c")
    sid = lax.axis_index("s")
    wid = sid * NC + cid
    zero16 = jnp.zeros((16,), jnp.float32)

    pltpu.sync_copy(s_hbm, s_v)
    pltpu.sync_copy(d_hbm, d_v)

    def zden(i, _):
        den_v[i, pl.ds(0, 16)] = zero16
        return 0
    lax.fori_loop(0, NPR, zden, 0, unroll=8)

    def zt(i, _):
        ztmp_v[i, pl.ds(0, 16)] = zero16
        return 0
    lax.fori_loop(0, DDZ, zt, 0, unroll=8)
    pltpu.sync_copy(ztmp_v, dden_shr.at[pl.ds(sid * DDZ, DDZ)])
    plsc.subcore_barrier()

    def sc_major(mj, _):
        m = NC * sid + mj

        def sc_chunk(ci, _):
            e0 = (m * WROWS + ci * SCH) * RW
            pltpu.sync_copy(src_hbm.at[pl.ds(e0, SCH * RW)], esrc_v)
            pltpu.sync_copy(dst3_hbm.at[m, pl.ds(ci * SCH, SCH)], edst_v)
            pltpu.sync_copy(ea_hbm.at[pl.ds(e0, SCH * RW)], eea_v)

            def rowloop(r, _):
                for k in range(RW // 16):
                    o = (r * (RW // 16) + k) * 16
                    src16 = esrc_v[pl.ds(o, 16)]
                    dst16 = edst_v.at[r][pl.ds(k * 16, 16)]
                    a = (plsc.load_gather(s_v, [src16])
                         + plsc.load_gather(d_v, [dst16])
                         + eea_v[pl.ds(o, 16)])
                    a = jnp.maximum(a, 0.2 * a)
                    plsc.addupdate_scatter(
                        den_v,
                        [jnp.right_shift(dst16, 4), jnp.bitwise_and(dst16, 15)],
                        jnp.exp(a))
                return 0
            lax.fori_loop(0, SCH, rowloop, 0, unroll=2)
            return 0
        lax.fori_loop(0, WROWS // SCH, sc_chunk, 0)
        return 0
    lax.fori_loop(0, NR // (NS * WROWS), sc_major, 0)

    iota16 = lax.iota(jnp.int32, 16)
    for c in range(NPR // 128):
        for g in range(8):
            idxb_v[pl.ds(g * 16, 16)] = iota16 + (c * 128 + g * 16)
        pltpu.sync_copy(den_v.at[pl.ds(c * 128, 128)],
                        dden_shr.at[idxb_v], add=True)
    plsc.subcore_barrier()
    pltpu.sync_copy(dden_shr, den_v)

    def cf_chunk(ci, _):
        e0 = (wid * WROWS + ci * SCH) * RW
        pltpu.sync_copy(src_hbm.at[pl.ds(e0, SCH * RW)], esrc_v)
        pltpu.sync_copy(dst3_hbm.at[wid, pl.ds(ci * SCH, SCH)], edst_v)
        pltpu.sync_copy(ea_hbm.at[pl.ds(e0, SCH * RW)], eea_v)

        @plsc.parallel_loop(0, SCH, unroll=2)
        def cf_rowloop(r):
            for k in range(RW // 16):
                o = (r * (RW // 16) + k) * 16
                src16 = esrc_v[pl.ds(o, 16)]
                dst16 = edst_v.at[r][pl.ds(k * 16, 16)]
                a = (plsc.load_gather(s_v, [src16])
                     + plsc.load_gather(d_v, [dst16])
                     + eea_v[pl.ds(o, 16)])
                a = jnp.maximum(a, 0.2 * a)
                den16 = plsc.load_gather(
                    den_v,
                    [jnp.right_shift(dst16, 4), jnp.bitwise_and(dst16, 15)])
                ccoef_v[pl.ds(o, 16)] = jnp.exp(a) / (den16 + 1e-16)
        pltpu.sync_copy(ccoef_v, coef_hbm.at[pl.ds(e0, SCH * RW)])
        return 0
    lax.fori_loop(0, WROWS // SCH, cf_chunk, 0)


def _coef_sc(src, dst3, ea, s, d):
    mesh = plsc.VectorSubcoreMesh(core_axis_name="c", subcore_axis_name="s",
                                  num_cores=NC, num_subcores=NS)
    call = pl.kernel(
        _coef_sc_body,
        out_type=jax.ShapeDtypeStruct((E,), jnp.float32),
        mesh=mesh,
        compiler_params=pltpu.CompilerParams(needs_layout_passes=False,
                                             use_tc_tiling_on_sc=False),
        scratch_types=[
            pltpu.VMEM((N,), jnp.float32),
            pltpu.VMEM((N,), jnp.float32),
            pltpu.VMEM((NPR, 16), jnp.float32),
            pltpu.VMEM((SCH * RW,), jnp.int32),
            pltpu.VMEM((SCH, RW), jnp.int32),
            pltpu.VMEM((SCH * RW,), jnp.float32),
            pltpu.VMEM((SCH * RW,), jnp.float32),
            pltpu.VMEM((DDZ, 16), jnp.float32),
            pltpu.VMEM((128,), jnp.int32),
            pltpu.VMEM_SHARED((NPR, 16), jnp.float32),
        ],
    )
    return call(src, dst3, ea, s, d)


def _scat_sc_body(src_hbm, dst3_hbm, coef_hbm, hw_hbm, out_hbm,
                  wdst2_v, srcr_v, coefr_v, rows_v,
                  acc_shr, semg, sems, semi):
    cid = lax.axis_index("c")
    sid = lax.axis_index("s")
    wid = sid * NC + cid
    zero16 = jnp.zeros((16,), jnp.float32)
    nbase = sid * NZT

    pltpu.sync_copy(dst3_hbm.at[wid], wdst2_v)

    def zrow(i, _):
        row = rows_v.at[0].at[i]
        for j in range(H // 16):
            row[pl.ds(j * 16, 16)] = zero16
        return 0
    lax.fori_loop(0, RW, zrow, 0)
    for p in range(NZT // RW):
        pltpu.sync_copy(rows_v.at[0], acc_shr.at[pl.ds(nbase + p * RW, RW)])
    plsc.subcore_barrier()

    ebase = wid * WROWS * RW

    def fire_smalls(r, b):
        pltpu.async_copy(src_hbm.at[pl.ds(ebase + r * RW, RW)],
                         srcr_v.at[b], semi.at[b])
        pltpu.async_copy(coef_hbm.at[pl.ds(ebase + r * RW, RW)],
                         coefr_v.at[b], semi.at[b])

    def wait_smalls(r, b):
        pltpu.make_async_copy(src_hbm.at[pl.ds(ebase + r * RW, RW)],
                              srcr_v.at[b], semi.at[b]).wait()
        pltpu.make_async_copy(coef_hbm.at[pl.ds(ebase + r * RW, RW)],
                              coefr_v.at[b], semi.at[b]).wait()

    def fire_gather(r, b):
        pltpu.async_copy(hw_hbm.at[srcr_v.at[b]], rows_v.at[b], semg.at[b])

    def wait_gather(r, b):
        pltpu.make_async_copy(hw_hbm.at[srcr_v.at[b]], rows_v.at[b],
                              semg.at[b]).wait()

    def fire_scatter(r, b):
        pltpu.async_copy(rows_v.at[b], acc_shr.at[wdst2_v.at[r]],
                         sems.at[b], add=True)

    def wait_scatter(r, b):
        pltpu.make_async_copy(rows_v.at[b], acc_shr.at[wdst2_v.at[r]],
                              sems.at[b]).wait()

    def step(r):
        b = r % NBUF

        @pl.when(r >= 2)
        def _():
            wait_scatter(r - 2, (r + 1) % NBUF)

        @pl.when(r + 2 < WROWS)
        def _():
            fire_smalls(r + 2, (r + 2) % NBUF)

        @pl.when(r + 1 < WROWS)
        def _():
            wait_smalls(r + 1, (r + 1) % NBUF)
            fire_gather(r + 1, (r + 1) % NBUF)

        wait_gather(r, b)

        @plsc.parallel_loop(0, RW, unroll=4)
        def scale(e):
            c16 = plsc.load_gather(coefr_v.at[b],
                                   [jnp.full((16,), e, jnp.int32)])
            row = rows_v.at[b].at[e]
            for j in range(H // 16):
                row[pl.ds(j * 16, 16)] = row[pl.ds(j * 16, 16)] * c16
        fire_scatter(r, b)

    fire_smalls(0, 0)
    fire_smalls(1, 1)
    wait_smalls(0, 0)
    fire_gather(0, 0)

    def steps3(i, _):
        for bb in range(NBUF):
            step(i * NBUF + bb)
        return 0
    nfull = (WROWS // NBUF) * NBUF
    lax.fori_loop(0, WROWS // NBUF, steps3, 0)
    for r in range(nfull, WROWS):
        step(r)
    for r in range(WROWS - 2, WROWS):
        wait_scatter(r, r % NBUF)
    plsc.subcore_barrier()

    for p in range(NZT // RW):
        pltpu.sync_copy(acc_shr.at[pl.ds(nbase + p * RW, RW)], rows_v.at[0])
        pltpu.sync_copy(rows_v.at[0], out_hbm.at[cid, pl.ds(nbase + p * RW, RW)])


def _scat_sc(src, dst3, coef, hw):
    mesh = plsc.VectorSubcoreMesh(core_axis_name="c", subcore_axis_name="s",
                                  num_cores=NC, num_subcores=NS)
    call = pl.kernel(
        _scat_sc_body,
        out_type=jax.ShapeDtypeStruct((NC, NP, H), jnp.float32),
        mesh=mesh,
        compiler_params=pltpu.CompilerParams(needs_layout_passes=False,
                                             use_tc_tiling_on_sc=False),
        scratch_types=[
            pltpu.VMEM((WROWS, RW), jnp.int32),
            pltpu.VMEM((NBUF, RW), jnp.int32),
            pltpu.VMEM((NBUF, RW), jnp.float32),
            pltpu.VMEM((NBUF, RW, H), jnp.float32),
            pltpu.VMEM_SHARED((NP, H), jnp.float32),
            pltpu.SemaphoreType.DMA((NBUF,)),
            pltpu.SemaphoreType.DMA((NBUF,)),
            pltpu.SemaphoreType.DMA((NBUF,)),
        ],
    )
    return call(src, dst3, coef, hw)


def _gat_sc(src, dst3, ea, s, d, hw):
    coef = _coef_sc(src, dst3, ea, s, d)
    return _scat_sc(src, dst3, coef, hw)


def kernel(x, edge_index, edge_attr, batch, W_ne, b_ne, ln_g, ln_b, W_ee, b_ee,
           W1, We1, as1, ad1, ae1, b1, W2, We2, as2, ad2, ae2, b2, Wg, bg,
           Wq1, bq1, Wq2, bq2):
    src = edge_index[0]
    dst = edge_index[1]
    dst2 = dst.reshape(NW, WROWS, RW)

    hw1, s1, d1 = _tc_prep(x, W_ne, b_ne, ln_g, ln_b, W1, as1, ad1)
    ea1, ea2 = _tc_edge(edge_attr, W_ee, b_ee, We1, ae1, We2, ae2)

    p1 = _gat_sc(src, dst2, ea1, s1.reshape(N), d1.reshape(N), hw1)
    hw2, s2, d2 = _tc_mid(p1[:, :N, :], b1, W2, as2, ad2)

    p2 = _gat_sc(src, dst2, ea2, s2.reshape(N), d2.reshape(N), hw2)
    q = _tc_final(p2[:, :N, :], b2, Wg, bg, Wq1, bq1, Wq2, bq2)
    return q.reshape(N)


# speedup vs baseline: 30.0305x; 1.0930x over previous
import functools

import jax
import jax.numpy as jnp
from jax import lax
from jax.experimental import pallas as pl
from jax.experimental.pallas import tpu as pltpu
from jax.experimental.pallas import tpu_sc as plsc

N = 10000
E = 320000
DF = 128
DE = 16
H = 128

NP = 10240
NC, NS = 2, 16
NW = NC * NS
RW = 80
NR = E // RW
SROWS = NR // NS
SCH = 125
WROWS = NR // NW
NBUF = 3
NZT = NP // NS
NPR = NP // 16
DDZ = NPR // NS


def _dot3(a, b):
    return jnp.dot(a, b, preferred_element_type=jnp.float32)


def _prep_body(x_ref, wne_ref, bne_ref, g_ref, b_ref, w1_ref, as_ref, ad_ref,
               hw_ref, s_ref, d_ref):
    h = jnp.maximum(_dot3(x_ref[...], wne_ref[...]) + bne_ref[...][None, :],
                    0.0)
    mu = jnp.mean(h, axis=1, keepdims=True)
    var = jnp.mean((h - mu) ** 2, axis=1, keepdims=True)
    h = (h - mu) / jnp.sqrt(var + 1e-5) * g_ref[...][None, :] + b_ref[...][None, :]
    hw = _dot3(h, w1_ref[...])
    hw_ref[...] = hw
    s_ref[...] = jnp.sum(hw * as_ref[...][None, :], axis=1, keepdims=True)
    d_ref[...] = jnp.sum(hw * ad_ref[...][None, :], axis=1, keepdims=True)


def _tc_prep(x, wne, bne, g, b, w1, a_s, a_d):
    return pl.pallas_call(
        _prep_body,
        out_shape=[jax.ShapeDtypeStruct((N, H), jnp.float32),
                   jax.ShapeDtypeStruct((N, 1), jnp.float32),
                   jax.ShapeDtypeStruct((N, 1), jnp.float32)],
    )(x, wne, bne, g, b, w1, a_s, a_d)


def _mid_body(p_ref, b1_ref, w2_ref, as_ref, ad_ref, hw_ref, s_ref, d_ref):
    h = jnp.maximum(p_ref[0][:N] + p_ref[1][:N] + b1_ref[...][None, :], 0.0)
    hw = _dot3(h, w2_ref[...])
    hw_ref[...] = hw
    s_ref[...] = jnp.sum(hw * as_ref[...][None, :], axis=1, keepdims=True)
    d_ref[...] = jnp.sum(hw * ad_ref[...][None, :], axis=1, keepdims=True)


def _tc_mid(partials, b1, w2, a_s, a_d):
    return pl.pallas_call(
        _mid_body,
        out_shape=[jax.ShapeDtypeStruct((N, H), jnp.float32),
                   jax.ShapeDtypeStruct((N, 1), jnp.float32),
                   jax.ShapeDtypeStruct((N, 1), jnp.float32)],
    )(partials, b1, w2, a_s, a_d)


EB = 12800


def _edge_body(ea_ref, wee_ref, bee_ref, we1_ref, ae1_ref, we2_ref, ae2_ref,
               o1_ref, o2_ref):
    f = jnp.float32
    e = jnp.maximum(
        jnp.dot(ea_ref[...], wee_ref[...], preferred_element_type=f)
        + bee_ref[...][None, :], 0.0)
    ee1 = jnp.dot(e, we1_ref[...], preferred_element_type=f)
    ee2 = jnp.dot(e, we2_ref[...], preferred_element_type=f)
    o1_ref[...] = jnp.sum(ee1 * ae1_ref[...][None, :], axis=1).reshape(
        1, EB // 128, 128)
    o2_ref[...] = jnp.sum(ee2 * ae2_ref[...][None, :], axis=1).reshape(
        1, EB // 128, 128)


def _tc_edge(edge_attr, wee, bee, we1, ae1, we2, ae2):
    nb = E // EB
    o1, o2 = pl.pallas_call(
        _edge_body,
        grid=(nb,),
        in_specs=[
            pl.BlockSpec((EB, DE), lambda i: (i, 0)),
            pl.BlockSpec((DE, H), lambda i: (0, 0)),
            pl.BlockSpec((H,), lambda i: (0,)),
            pl.BlockSpec((H, H), lambda i: (0, 0)),
            pl.BlockSpec((H,), lambda i: (0,)),
            pl.BlockSpec((H, H), lambda i: (0, 0)),
            pl.BlockSpec((H,), lambda i: (0,)),
        ],
        out_specs=[pl.BlockSpec((1, EB // 128, 128), lambda i: (i, 0, 0)),
                   pl.BlockSpec((1, EB // 128, 128), lambda i: (i, 0, 0))],
        out_shape=[jax.ShapeDtypeStruct((nb, EB // 128, 128), jnp.float32),
                   jax.ShapeDtypeStruct((nb, EB // 128, 128), jnp.float32)],
    )(edge_attr, wee, bee, we1, ae1, we2, ae2)
    return o1.reshape(E), o2.reshape(E)


def _final_body(p_ref, b2_ref, wg_ref, bg_ref, wq1_ref, bq1_ref, wq2_ref,
                bq2_ref, q_ref):
    h = jnp.maximum(p_ref[0][:N] + p_ref[1][:N] + b2_ref[...][None, :], 0.0)
    gate = (jnp.dot(h, wg_ref[...], preferred_element_type=jnp.float32)
            + bg_ref[...][None, :])[:, 0]
    gm = jnp.max(gate)
    gex = jnp.exp(gate - gm)
    gden = jnp.sum(gex)
    coef = gex / (gden + 1e-16)
    pooled = jnp.sum(h * coef[:, None], axis=0)
    zb = jnp.dot(pooled[None, :], wq1_ref[H:2 * H, :],
                 preferred_element_type=jnp.float32)
    z = _dot3(h, wq1_ref[0:H, :]) + zb + bq1_ref[...][None, :]
    z = jnp.maximum(z, 0.0)
    q_ref[...] = (jnp.dot(z, wq2_ref[...], preferred_element_type=jnp.float32)
                  + bq2_ref[...][None, :])


def _tc_final(partials, b2, wg, bg, wq1, bq1, wq2, bq2):
    return pl.pallas_call(
        _final_body,
        out_shape=jax.ShapeDtypeStruct((N, 1), jnp.float32),
    )(partials, b2, wg, bg, wq1, bq1, wq2, bq2)


def _coef_sc_body(src_hbm, dst3_hbm, ea_hbm, s_hbm, d_hbm, coef_hbm,
                  s_v, d_v, den_v, esrc_v, edst_v, eea_v, ccoef_v,
                  ztmp_v, idxb_v, dden_shr):
    cid = lax.axis_index("c")
    sid = lax.axis_index("s")
    wid = sid * NC + cid
    zero16 = jnp.zeros((16,), jnp.float32)

    pltpu.sync_copy(s_hbm, s_v)
    pltpu.sync_copy(d_hbm, d_v)

    def zden(i, _):
        den_v[i, pl.ds(0, 16)] = zero16
        return 0
    lax.fori_loop(0, NPR, zden, 0, unroll=8)

    def zt(i, _):
        ztmp_v[i, pl.ds(0, 16)] = zero16
        return 0
    lax.fori_loop(0, DDZ, zt, 0, unroll=8)
    pltpu.sync_copy(ztmp_v, dden_shr.at[pl.ds(sid * DDZ, DDZ)])
    plsc.subcore_barrier()

    def sc_major(mj, _):
        m = NC * sid + mj

        def sc_chunk(ci, _):
            e0 = (m * WROWS + ci * SCH) * RW
            pltpu.sync_copy(src_hbm.at[pl.ds(e0, SCH * RW)], esrc_v)
            pltpu.sync_copy(dst3_hbm.at[m, pl.ds(ci * SCH, SCH)], edst_v)
            pltpu.sync_copy(ea_hbm.at[pl.ds(e0, SCH * RW)], eea_v)

            def rowloop(r, _):
                for k in range(RW // 16):
                    o = (r * (RW // 16) + k) * 16
                    src16 = esrc_v[pl.ds(o, 16)]
                    dst16 = edst_v.at[r][pl.ds(k * 16, 16)]
                    a = (plsc.load_gather(s_v, [src16])
                         + plsc.load_gather(d_v, [dst16])
                         + eea_v[pl.ds(o, 16)])
                    a = jnp.maximum(a, 0.2 * a)
                    plsc.addupdate_scatter(
                        den_v,
                        [jnp.right_shift(dst16, 4), jnp.bitwise_and(dst16, 15)],
                        jnp.exp(a))
                return 0
            lax.fori_loop(0, SCH, rowloop, 0)
            return 0
        lax.fori_loop(0, WROWS // SCH, sc_chunk, 0)
        return 0
    lax.fori_loop(0, NR // (NS * WROWS), sc_major, 0)

    iota16 = lax.iota(jnp.int32, 16)
    for c in range(NPR // 128):
        for g in range(8):
            idxb_v[pl.ds(g * 16, 16)] = iota16 + (c * 128 + g * 16)
        pltpu.sync_copy(den_v.at[pl.ds(c * 128, 128)],
                        dden_shr.at[idxb_v], add=True)
    plsc.subcore_barrier()
    pltpu.sync_copy(dden_shr, den_v)

    def cf_chunk(ci, _):
        e0 = (wid * WROWS + ci * SCH) * RW
        pltpu.sync_copy(src_hbm.at[pl.ds(e0, SCH * RW)], esrc_v)
        pltpu.sync_copy(dst3_hbm.at[wid, pl.ds(ci * SCH, SCH)], edst_v)
        pltpu.sync_copy(ea_hbm.at[pl.ds(e0, SCH * RW)], eea_v)

        def cf_rowloop(r, _):
            for k in range(RW // 16):
                o = (r * (RW // 16) + k) * 16
                src16 = esrc_v[pl.ds(o, 16)]
                dst16 = edst_v.at[r][pl.ds(k * 16, 16)]
                a = (plsc.load_gather(s_v, [src16])
                     + plsc.load_gather(d_v, [dst16])
                     + eea_v[pl.ds(o, 16)])
                a = jnp.maximum(a, 0.2 * a)
                den16 = plsc.load_gather(
                    den_v,
                    [jnp.right_shift(dst16, 4), jnp.bitwise_and(dst16, 15)])
                ccoef_v[pl.ds(o, 16)] = jnp.exp(a) / (den16 + 1e-16)
            return 0
        lax.fori_loop(0, SCH, cf_rowloop, 0)
        pltpu.sync_copy(ccoef_v, coef_hbm.at[pl.ds(e0, SCH * RW)])
        return 0
    lax.fori_loop(0, WROWS // SCH, cf_chunk, 0)


def _coef_sc(src, dst3, ea, s, d):
    mesh = plsc.VectorSubcoreMesh(core_axis_name="c", subcore_axis_name="s",
                                  num_cores=NC, num_subcores=NS)
    call = pl.kernel(
        _coef_sc_body,
        out_type=jax.ShapeDtypeStruct((E,), jnp.float32),
        mesh=mesh,
        compiler_params=pltpu.CompilerParams(needs_layout_passes=False,
                                             use_tc_tiling_on_sc=False),
        scratch_types=[
            pltpu.VMEM((N,), jnp.float32),
            pltpu.VMEM((N,), jnp.float32),
            pltpu.VMEM((NPR, 16), jnp.float32),
            pltpu.VMEM((SCH * RW,), jnp.int32),
            pltpu.VMEM((SCH, RW), jnp.int32),
            pltpu.VMEM((SCH * RW,), jnp.float32),
            pltpu.VMEM((SCH * RW,), jnp.float32),
            pltpu.VMEM((DDZ, 16), jnp.float32),
            pltpu.VMEM((128,), jnp.int32),
            pltpu.VMEM_SHARED((NPR, 16), jnp.float32),
        ],
    )
    return call(src, dst3, ea, s, d)


def _scat_sc_body(src_hbm, dst3_hbm, coef_hbm, hw_hbm, out_hbm,
                  wdst2_v, srcr_v, coefr_v, rows_v,
                  acc_shr, semg, sems, semi):
    cid = lax.axis_index("c")
    sid = lax.axis_index("s")
    wid = sid * NC + cid
    zero16 = jnp.zeros((16,), jnp.float32)
    nbase = sid * NZT

    pltpu.sync_copy(dst3_hbm.at[wid], wdst2_v)

    def zrow(i, _):
        row = rows_v.at[0].at[i]
        for j in range(H // 16):
            row[pl.ds(j * 16, 16)] = zero16
        return 0
    lax.fori_loop(0, RW, zrow, 0)
    for p in range(NZT // RW):
        pltpu.sync_copy(rows_v.at[0], acc_shr.at[pl.ds(nbase + p * RW, RW)])
    plsc.subcore_barrier()

    ebase = wid * WROWS * RW

    def fire_smalls(r, b):
        pltpu.async_copy(src_hbm.at[pl.ds(ebase + r * RW, RW)],
                         srcr_v.at[b], semi.at[b])
        pltpu.async_copy(coef_hbm.at[pl.ds(ebase + r * RW, RW)],
                         coefr_v.at[b], semi.at[b])

    def wait_smalls(r, b):
        pltpu.make_async_copy(src_hbm.at[pl.ds(ebase + r * RW, RW)],
                              srcr_v.at[b], semi.at[b]).wait()
        pltpu.make_async_copy(coef_hbm.at[pl.ds(ebase + r * RW, RW)],
                              coefr_v.at[b], semi.at[b]).wait()

    def fire_gather(r, b):
        pltpu.async_copy(hw_hbm.at[srcr_v.at[b]], rows_v.at[b], semg.at[b])

    def wait_gather(r, b):
        pltpu.make_async_copy(hw_hbm.at[srcr_v.at[b]], rows_v.at[b],
                              semg.at[b]).wait()

    def fire_scatter(r, b):
        pltpu.async_copy(rows_v.at[b], acc_shr.at[wdst2_v.at[r]],
                         sems.at[b], add=True)

    def wait_scatter(r, b):
        pltpu.make_async_copy(rows_v.at[b], acc_shr.at[wdst2_v.at[r]],
                              sems.at[b]).wait()

    def step(r):
        b = r % NBUF

        @pl.when(r >= 2)
        def _():
            wait_scatter(r - 2, (r + 1) % NBUF)

        @pl.when(r + 2 < WROWS)
        def _():
            fire_smalls(r + 2, (r + 2) % NBUF)

        @pl.when(r + 1 < WROWS)
        def _():
            wait_smalls(r + 1, (r + 1) % NBUF)
            fire_gather(r + 1, (r + 1) % NBUF)

        wait_gather(r, b)

        def scale(e, _):
            c16 = plsc.load_gather(coefr_v.at[b],
                                   [jnp.full((16,), e, jnp.int32)])
            row = rows_v.at[b].at[e]
            for j in range(H // 16):
                row[pl.ds(j * 16, 16)] = row[pl.ds(j * 16, 16)] * c16
            return 0
        lax.fori_loop(0, RW, scale, 0)
        fire_scatter(r, b)

    fire_smalls(0, 0)
    fire_smalls(1, 1)
    wait_smalls(0, 0)
    fire_gather(0, 0)

    def steps3(i, _):
        for bb in range(NBUF):
            step(i * NBUF + bb)
        return 0
    nfull = (WROWS // NBUF) * NBUF
    lax.fori_loop(0, WROWS // NBUF, steps3, 0)
    for r in range(nfull, WROWS):
        step(r)
    for r in range(WROWS - 2, WROWS):
        wait_scatter(r, r % NBUF)
    plsc.subcore_barrier()

    for p in range(NZT // RW):
        pltpu.sync_copy(acc_shr.at[pl.ds(nbase + p * RW, RW)], rows_v.at[0])
        pltpu.sync_copy(rows_v.at[0], out_hbm.at[cid, pl.ds(nbase + p * RW, RW)])


def _scat_sc(src, dst3, coef, hw):
    mesh = plsc.VectorSubcoreMesh(core_axis_name="c", subcore_axis_name="s",
                                  num_cores=NC, num_subcores=NS)
    call = pl.kernel(
        _scat_sc_body,
        out_type=jax.ShapeDtypeStruct((NC, NP, H), jnp.float32),
        mesh=mesh,
        compiler_params=pltpu.CompilerParams(needs_layout_passes=False,
                                             use_tc_tiling_on_sc=False),
        scratch_types=[
            pltpu.VMEM((WROWS, RW), jnp.int32),
            pltpu.VMEM((NBUF, RW), jnp.int32),
            pltpu.VMEM((NBUF, RW), jnp.float32),
            pltpu.VMEM((NBUF, RW, H), jnp.float32),
            pltpu.VMEM_SHARED((NP, H), jnp.float32),
            pltpu.SemaphoreType.DMA((NBUF,)),
            pltpu.SemaphoreType.DMA((NBUF,)),
            pltpu.SemaphoreType.DMA((NBUF,)),
        ],
    )
    return call(src, dst3, coef, hw)


def _gat_sc(src, dst3, ea, s, d, hw):
    coef = _coef_sc(src, dst3, ea, s, d)
    return _scat_sc(src, dst3, coef, hw)


def kernel(x, edge_index, edge_attr, batch, W_ne, b_ne, ln_g, ln_b, W_ee, b_ee,
           W1, We1, as1, ad1, ae1, b1, W2, We2, as2, ad2, ae2, b2, Wg, bg,
           Wq1, bq1, Wq2, bq2):
    src = edge_index[0]
    dst = edge_index[1]
    dst2 = dst.reshape(NW, WROWS, RW)

    hw1, s1, d1 = _tc_prep(x, W_ne, b_ne, ln_g, ln_b, W1, as1, ad1)
    ea1, ea2 = _tc_edge(edge_attr, W_ee, b_ee, We1, ae1, We2, ae2)

    p1 = _gat_sc(src, dst2, ea1, s1.reshape(N), d1.reshape(N), hw1)
    hw2, s2, d2 = _tc_mid(p1, b1, W2, as2, ad2)

    p2 = _gat_sc(src, dst2, ea2, s2.reshape(N), d2.reshape(N), hw2)
    q = _tc_final(p2, b2, Wg, bg, Wq1, bq1, Wq2, bq2)
    return q.reshape(N)
